# baseline (speedup 1.0000x reference)
import functools

import jax
import jax.numpy as jnp
from jax import lax
from jax.experimental import pallas as pl
from jax.experimental.pallas import tpu as pltpu
from jax.experimental.pallas import tpu_sc as plsc

D = 16
NV = 1000000
GW = 128
RPG = GW // D
NG = NV // RPG
B = 16384
NC = 2
NS = 16
NW = NC * NS
BPW = B // NW
HP = BPW // 2
CH = 128
NCH = HP // CH

GB = 1248
CPB = GB * RPG
TCN = -(-NV // CPB)
NGP = TCN * GB

_mesh = plsc.VectorSubcoreMesh(core_axis_name="c", subcore_axis_name="s")


def _repack_body(in_ref, out_ref):
    b = in_ref[...]
    t = jnp.transpose(b)
    t3 = t.reshape(GB, RPG, D)
    for s in range(RPG):
        out_ref[:, s * D:(s + 1) * D] = t3[:, s, :]


_repack = pl.pallas_call(
    _repack_body,
    out_shape=jax.ShapeDtypeStruct((NGP, GW), jnp.float32),
    grid=(TCN,),
    in_specs=[pl.BlockSpec((D, CPB), lambda i: (0, i))],
    out_specs=pl.BlockSpec((GB, GW), lambda i: (i, 0)),
)


@functools.partial(
    pl.kernel,
    out_type=jax.ShapeDtypeStruct((B,), jnp.float32),
    mesh=_mesh,
    compiler_params=pltpu.CompilerParams(needs_layout_passes=False),
    scratch_types=[
        pltpu.VMEM((BPW,), jnp.int32),
        pltpu.VMEM((BPW,), jnp.int32),
        pltpu.VMEM((BPW,), jnp.int32),
        pltpu.VMEM((BPW,), jnp.int32),
        pltpu.VMEM((HP, GW), jnp.float32),
        pltpu.VMEM((HP, GW), jnp.float32),
        pltpu.VMEM((BPW,), jnp.float32),
        pltpu.SemaphoreType.DMA,
    ],
)
def _skipgram_sc(ci_hbm, xi_hbm, cw_hbm, xw_hbm, out_hbm,
                 ci_v, xi_v, cg_v, xg_v, v_v, u_v, o_v, sem):
    wid = lax.axis_index("s") * NC + lax.axis_index("c")
    base = wid * BPW

    pltpu.sync_copy(ci_hbm.at[pl.ds(base, BPW)], ci_v)
    pltpu.sync_copy(xi_hbm.at[pl.ds(base, BPW)], xi_v)

    def gid_body(t, carry):
        sl = pl.ds(t * 16, 16)
        cg_v[sl] = lax.shift_right_logical(ci_v[sl], 3)
        xg_v[sl] = lax.shift_right_logical(xi_v[sl], 3)
        return carry

    lax.fori_loop(0, BPW // 16, gid_body, 0)

    lanes = lax.iota(jnp.int32, 16)

    for h in range(2):
        hbase = h * HP
        copies = []
        for c in range(NCH):
            ssl = pl.ds(hbase + c * CH, CH)
            dsl = pl.ds(c * CH, CH)
            copies.append(
                pltpu.async_copy(cw_hbm.at[cg_v.at[ssl]], v_v.at[dsl], sem))
            copies.append(
                pltpu.async_copy(xw_hbm.at[xg_v.at[ssl]], u_v.at[dsl], sem))
        for cp in copies:
            cp.wait()

        def chunk_body(k, carry):
            isl = pl.ds(hbase + k * 16, 16)
            srow = (ci_v[isl] & 7) * D
            urow = (xi_v[isl] & 7) * D
            prow = k * 16 + lanes
            acc = jnp.zeros((16,), jnp.float32)
            for d in range(D):
                cv = plsc.load_gather(v_v, [prow, srow + d])
                cu = plsc.load_gather(u_v, [prow, urow + d])
                acc = acc + cv * cu
            o_v[pl.ds(hbase + k * 16, 16)] = acc
            return carry

        lax.fori_loop(0, HP // 16, chunk_body, 0)

    pltpu.sync_copy(o_v, out_hbm.at[pl.ds(base, BPW)])


def kernel(tc_center, tc_context, center_weight, context_weight):
    cw = _repack(jnp.transpose(center_weight))
    xw = _repack(jnp.transpose(context_weight))
    return _skipgram_sc(tc_center, tc_context, cw, xw)
